# baseline (speedup 1.0000x reference)
import functools

import jax
import jax.numpy as jnp
from jax import lax
from jax.experimental import pallas as pl
from jax.experimental.pallas import tpu as pltpu
from jax.experimental.pallas import tpu_sc as plsc

N_NODES = 10000
N_EDGES = 320000
F = 128
L = 16

NC = 2
NS = 16
NPT = 632
NPAD = NS * NPT
E_PER_SC = N_EDGES // NC
CE = 2000
MB = 2176
GB = 128


_ROWS_BLK = 2000


def _mm_kernel(x_ref, tw_ref, pw_ref, tb_ref, pb_ref, a_ref, b_ref):
    x = x_ref[...]
    tw = tw_ref[...]
    dn = (((1,), (1,)), ((), ()))
    a_ref[...] = (
        lax.dot_general(x, tw, dn, preferred_element_type=jnp.float32)
        + tb_ref[...] + pb_ref[...]
    )
    b_ref[...] = lax.dot_general(
        x, pw_ref[...] - tw, dn, preferred_element_type=jnp.float32
    )


def _node_matmuls(x, theta_w, phi_w, theta_b, phi_b):
    n = x.shape[0]
    grid = n // _ROWS_BLK
    return pl.pallas_call(
        _mm_kernel,
        grid=(grid,),
        in_specs=[
            pl.BlockSpec((_ROWS_BLK, F), lambda i: (i, 0)),
            pl.BlockSpec((F, F), lambda i: (0, 0)),
            pl.BlockSpec((F, F), lambda i: (0, 0)),
            pl.BlockSpec((1, F), lambda i: (0, 0)),
            pl.BlockSpec((1, F), lambda i: (0, 0)),
        ],
        out_specs=[
            pl.BlockSpec((_ROWS_BLK, F), lambda i: (i, 0)),
            pl.BlockSpec((_ROWS_BLK, F), lambda i: (i, 0)),
        ],
        out_shape=[
            jax.ShapeDtypeStruct((n, F), jnp.float32),
            jax.ShapeDtypeStruct((n, F), jnp.float32),
        ],
    )(x, theta_w, phi_w, theta_b, phi_b)


def _sc_body(b_hbm, src_hbm, dst_hbm, out_hbm, c_loc, src_ch, dst_ch, src_m,
             dst_m, rows, sem):
    c = lax.axis_index("c")
    s = lax.axis_index("s")
    lo = s * NPT

    neg_inf = jnp.full((L,), -jnp.inf, dtype=jnp.float32)

    def init_row(r, carry):
        for j in range(F // L):
            c_loc[r, pl.ds(j * L, L)] = neg_inf
        return carry

    lax.fori_loop(0, NPT + 1, init_row, 0)

    zero_v = jnp.zeros((L,), dtype=jnp.int32)

    def init_idx(g, carry):
        src_m[pl.ds(g * L, L)] = zero_v
        return carry

    lax.fori_loop(0, MB // L, init_idx, 0)

    lane = lax.iota(jnp.int32, L)
    sentv = jnp.full((L,), NPT, dtype=jnp.int32)
    ebase = c * E_PER_SC

    def chunk_body(k, carry):
        off = ebase + k * CE
        pltpu.sync_copy(src_hbm.at[pl.ds(off, CE)], src_ch)
        pltpu.sync_copy(dst_hbm.at[pl.ds(off, CE)], dst_ch)

        def scan_g(g, cnt):
            dv = dst_ch[pl.ds(g * L, L)]
            sv = src_ch[pl.ds(g * L, L)]
            dl = dv - lo
            msk = (dl >= 0) & (dl < NPT)
            inc = plsc.cumsum(msk.astype(jnp.int32))
            pos = cnt + inc - 1
            plsc.store_scatter(src_m, [pos], sv, mask=msk)
            plsc.store_scatter(dst_m, [pos], dl, mask=msk)
            return cnt + jnp.max(inc)

        cnt = lax.fori_loop(0, CE // L, scan_g, 0)

        for t in range(GB // L):
            plsc.store_scatter(dst_m, [cnt + lane + t * L], sentv)

        nb = (cnt + GB - 1) // GB

        def batch_body(b, carry):
            pltpu.async_copy(
                b_hbm.at[src_m.at[pl.ds(b * GB, GB)]], rows, sem
            ).wait()

            def group_body(g, carry2):
                e0 = b * GB + g * L
                dv = dst_m[pl.ds(e0, L)]
                for i in range(L):
                    bc = jnp.take_along_axis(
                        dv,
                        jnp.full((L,), i, dtype=jnp.int32),
                        axis=0,
                        mode="promise_in_bounds",
                    )
                    for j in range(F // L):
                        colv = lane + (j * L)
                        cur = plsc.load_gather(c_loc, [bc, colv])
                        rv = rows[g * L + i, pl.ds(j * L, L)]
                        plsc.store_scatter(
                            c_loc, [bc, colv], jnp.maximum(cur, rv)
                        )
                return carry2

            lax.fori_loop(0, GB // L, group_body, 0)
            return carry

        lax.fori_loop(0, nb, batch_body, 0)
        return carry

    lax.fori_loop(0, E_PER_SC // CE, chunk_body, 0)

    pltpu.sync_copy(
        c_loc.at[pl.ds(0, NPT)], out_hbm.at[c, pl.ds(lo, NPT)]
    )


_sc_scatter_max = functools.partial(
    pl.kernel,
    out_type=jax.ShapeDtypeStruct((NC, NPAD, F), jnp.float32),
    mesh=plsc.VectorSubcoreMesh(core_axis_name="c", subcore_axis_name="s"),
    compiler_params=pltpu.CompilerParams(needs_layout_passes=False),
    scratch_types=[
        pltpu.VMEM((NPT + 1, F), jnp.float32),
        pltpu.VMEM((CE,), jnp.int32),
        pltpu.VMEM((CE,), jnp.int32),
        pltpu.VMEM((MB,), jnp.int32),
        pltpu.VMEM((MB,), jnp.int32),
        pltpu.VMEM((GB, F), jnp.float32),
        pltpu.SemaphoreType.DMA,
    ],
)(_sc_body)


def _combine_kernel(a_ref, c_ref, o_ref):
    cm = jnp.maximum(c_ref[0], c_ref[1])
    o_ref[...] = jnp.where(jnp.isfinite(cm), a_ref[...] + cm, 0.0)


def _combine(a, c_part):
    n = a.shape[0]
    grid = n // _ROWS_BLK
    return pl.pallas_call(
        _combine_kernel,
        grid=(grid,),
        in_specs=[
            pl.BlockSpec((_ROWS_BLK, F), lambda i: (i, 0)),
            pl.BlockSpec((NC, _ROWS_BLK, F), lambda i: (0, i, 0)),
        ],
        out_specs=pl.BlockSpec((_ROWS_BLK, F), lambda i: (i, 0)),
        out_shape=jax.ShapeDtypeStruct((n, F), jnp.float32),
    )(a, c_part)


@jax.jit
def kernel(h, edge_index, theta_w, theta_b, phi_w, phi_b):
    n_samples, n_points, n_dims = h.shape
    x = h.reshape(-1, n_dims)
    a, b = _node_matmuls(
        x,
        theta_w,
        phi_w,
        theta_b.reshape(1, F),
        phi_b.reshape(1, F),
    )
    c_part = _sc_scatter_max(b, edge_index[0], edge_index[1])
    out = _combine(a, c_part)
    return out.reshape(n_samples, n_points, F)


# speedup vs baseline: 4.3755x; 4.3755x over previous
import functools

import jax
import jax.numpy as jnp
from jax import lax
from jax.experimental import pallas as pl
from jax.experimental.pallas import tpu as pltpu
from jax.experimental.pallas import tpu_sc as plsc

N_NODES = 10000
N_EDGES = 320000
F = 128
L = 16

NC = 2
NS = 16
NPT = 632
NPAD = NS * NPT
E_PER_SC = N_EDGES // NC
CE = 4000
NCHUNK = E_PER_SC // CE
MB = 4096
GB = 64


_ROWS_BLK = 2000


def _mm_kernel(x_ref, tw_ref, pw_ref, tb_ref, pb_ref, a_ref, b_ref):
    x = x_ref[...]
    tw = tw_ref[...]
    dn = (((1,), (1,)), ((), ()))
    a_ref[...] = (
        lax.dot_general(x, tw, dn, preferred_element_type=jnp.float32)
        + tb_ref[...] + pb_ref[...]
    )
    b_ref[...] = lax.dot_general(
        x, pw_ref[...] - tw, dn, preferred_element_type=jnp.float32
    )


def _node_matmuls(x, theta_w, phi_w, theta_b, phi_b):
    n = x.shape[0]
    grid = n // _ROWS_BLK
    return pl.pallas_call(
        _mm_kernel,
        grid=(grid,),
        in_specs=[
            pl.BlockSpec((_ROWS_BLK, F), lambda i: (i, 0)),
            pl.BlockSpec((F, F), lambda i: (0, 0)),
            pl.BlockSpec((F, F), lambda i: (0, 0)),
            pl.BlockSpec((1, F), lambda i: (0, 0)),
            pl.BlockSpec((1, F), lambda i: (0, 0)),
        ],
        out_specs=[
            pl.BlockSpec((_ROWS_BLK, F), lambda i: (i, 0)),
            pl.BlockSpec((_ROWS_BLK, F), lambda i: (i, 0)),
        ],
        out_shape=[
            jax.ShapeDtypeStruct((n, F), jnp.float32),
            jax.ShapeDtypeStruct((n, F), jnp.float32),
        ],
    )(x, theta_w, phi_w, theta_b, phi_b)


def _sc_body(b_hbm, src_hbm, dst_hbm, out_hbm, c_loc,
             src_ch0, dst_ch0, src_ch1, dst_ch1, src_m, dst_m,
             rows0, rows1, sem_s0, sem_d0, sem_s1, sem_d1, sem_g0, sem_g1):
    c = lax.axis_index("c")
    s = lax.axis_index("s")
    lo = s * NPT

    src_ch = (src_ch0, src_ch1)
    dst_ch = (dst_ch0, dst_ch1)
    sem_s = (sem_s0, sem_s1)
    sem_d = (sem_d0, sem_d1)
    rows = (rows0, rows1)
    sem_g = (sem_g0, sem_g1)

    neg_inf = jnp.full((L,), -jnp.inf, dtype=jnp.float32)

    def init_row(r, carry):
        for j in range(F // L):
            c_loc[r, pl.ds(j * L, L)] = neg_inf
        return carry

    lax.fori_loop(0, NPT + 1, init_row, 0)

    zero_v = jnp.zeros((L,), dtype=jnp.int32)

    def init_idx(g, carry):
        src_m[pl.ds(g * L, L)] = zero_v
        return carry

    lax.fori_loop(0, MB // L, init_idx, 0)

    lane = lax.iota(jnp.int32, L)
    sentv = jnp.full((L,), NPT, dtype=jnp.int32)
    ebase = c * E_PER_SC

    def fire_chunk(k, buf):
        off = ebase + k * CE
        pltpu.async_copy(src_hbm.at[pl.ds(off, CE)], src_ch[buf], sem_s[buf])
        pltpu.async_copy(dst_hbm.at[pl.ds(off, CE)], dst_ch[buf], sem_d[buf])

    def wait_chunk(k, buf):
        off = ebase + k * CE
        pltpu.make_async_copy(
            src_hbm.at[pl.ds(off, CE)], src_ch[buf], sem_s[buf]
        ).wait()
        pltpu.make_async_copy(
            dst_hbm.at[pl.ds(off, CE)], dst_ch[buf], sem_d[buf]
        ).wait()

    def fire_gather(b, rbuf):
        pltpu.async_copy(
            b_hbm.at[src_m.at[pl.ds(b * GB, GB)]], rows[rbuf], sem_g[rbuf]
        )

    def wait_gather(b, rbuf):
        pltpu.make_async_copy(
            b_hbm.at[src_m.at[pl.ds(b * GB, GB)]], rows[rbuf], sem_g[rbuf]
        ).wait()

    def process_batch(b, rbuf):
        rbuf_ref = rows[rbuf]

        def group_body(g, carry2):
            e0 = b * GB + g * L
            dv = dst_m[pl.ds(e0, L)]
            for i in range(L):
                bc = jnp.take_along_axis(
                    dv,
                    jnp.full((L,), i, dtype=jnp.int32),
                    axis=0,
                    mode="promise_in_bounds",
                )
                for j in range(F // L):
                    colv = lane + (j * L)
                    cur = plsc.load_gather(c_loc, [bc, colv])
                    rv = rbuf_ref[g * L + i, pl.ds(j * L, L)]
                    plsc.store_scatter(
                        c_loc, [bc, colv], jnp.maximum(cur, rv)
                    )
            return carry2

        lax.fori_loop(0, GB // L, group_body, 0)

    def do_chunk(k, buf):
        wait_chunk(k, buf)

        def scan_g(g, cnt):
            dv = dst_ch[buf][pl.ds(g * L, L)]
            sv = src_ch[buf][pl.ds(g * L, L)]
            dl = dv - lo
            msk = dl.astype(jnp.uint32) < jnp.uint32(NPT)
            inc = plsc.cumsum(msk.astype(jnp.int32))
            pos = cnt + inc - 1
            plsc.store_scatter(src_m, [pos], sv, mask=msk)
            plsc.store_scatter(dst_m, [pos], dl, mask=msk)
            return cnt + jnp.max(inc)

        cnt = lax.fori_loop(0, CE // L, scan_g, 0)

        for t in range(GB // L):
            plsc.store_scatter(dst_m, [cnt + lane + t * L], sentv)

        nb = (cnt + GB - 1) // GB

        @pl.when(nb > 0)
        def _():
            fire_gather(0, 0)

            def pair_body(p, carry):
                b0 = 2 * p

                @pl.when(b0 + 1 < nb)
                def _():
                    fire_gather(b0 + 1, 1)

                wait_gather(b0, 0)
                process_batch(b0, 0)

                @pl.when(b0 + 1 < nb)
                def _():
                    @pl.when(b0 + 2 < nb)
                    def _():
                        fire_gather(b0 + 2, 0)

                    wait_gather(b0 + 1, 1)
                    process_batch(b0 + 1, 1)

                return carry

            lax.fori_loop(0, (nb + 1) // 2, pair_body, 0)

    fire_chunk(0, 0)

    def pair_chunks(p, carry):
        a = 2 * p
        fire_chunk(a + 1, 1)
        do_chunk(a, 0)

        @pl.when(a + 2 < NCHUNK)
        def _():
            fire_chunk(a + 2, 0)

        do_chunk(a + 1, 1)
        return carry

    lax.fori_loop(0, NCHUNK // 2, pair_chunks, 0)

    pltpu.sync_copy(
        c_loc.at[pl.ds(0, NPT)], out_hbm.at[c, pl.ds(lo, NPT)]
    )


_sc_scatter_max = functools.partial(
    pl.kernel,
    out_type=jax.ShapeDtypeStruct((NC, NPAD, F), jnp.float32),
    mesh=plsc.VectorSubcoreMesh(core_axis_name="c", subcore_axis_name="s"),
    compiler_params=pltpu.CompilerParams(needs_layout_passes=False),
    scratch_types=[
        pltpu.VMEM((NPT + 1, F), jnp.float32),
        pltpu.VMEM((CE,), jnp.int32),
        pltpu.VMEM((CE,), jnp.int32),
        pltpu.VMEM((CE,), jnp.int32),
        pltpu.VMEM((CE,), jnp.int32),
        pltpu.VMEM((MB,), jnp.int32),
        pltpu.VMEM((MB,), jnp.int32),
        pltpu.VMEM((GB, F), jnp.float32),
        pltpu.VMEM((GB, F), jnp.float32),
        pltpu.SemaphoreType.DMA,
        pltpu.SemaphoreType.DMA,
        pltpu.SemaphoreType.DMA,
        pltpu.SemaphoreType.DMA,
        pltpu.SemaphoreType.DMA,
        pltpu.SemaphoreType.DMA,
    ],
)(_sc_body)


def _combine_kernel(a_ref, c_ref, o_ref):
    cm = jnp.maximum(c_ref[0], c_ref[1])
    o_ref[...] = jnp.where(jnp.isfinite(cm), a_ref[...] + cm, 0.0)


def _combine(a, c_part):
    n = a.shape[0]
    grid = n // _ROWS_BLK
    return pl.pallas_call(
        _combine_kernel,
        grid=(grid,),
        in_specs=[
            pl.BlockSpec((_ROWS_BLK, F), lambda i: (i, 0)),
            pl.BlockSpec((NC, _ROWS_BLK, F), lambda i: (0, i, 0)),
        ],
        out_specs=pl.BlockSpec((_ROWS_BLK, F), lambda i: (i, 0)),
        out_shape=jax.ShapeDtypeStruct((n, F), jnp.float32),
    )(a, c_part)


@jax.jit
def kernel(h, edge_index, theta_w, theta_b, phi_w, phi_b):
    n_samples, n_points, n_dims = h.shape
    x = h.reshape(-1, n_dims)
    a, b = _node_matmuls(
        x,
        theta_w,
        phi_w,
        theta_b.reshape(1, F),
        phi_b.reshape(1, F),
    )
    c_part = _sc_scatter_max(b, edge_index[0], edge_index[1])
    out = _combine(a, c_part)
    return out.reshape(n_samples, n_points, F)
